# baseline (speedup 1.0000x reference)
import dataclasses
import functools

import jax
import jax.numpy as jnp
from jax import lax
from jax.experimental import pallas as pl
from jax.experimental.pallas import tpu as pltpu
from jax.experimental.pallas import tpu_sc as plsc

_B = 16384
_C = 1000
_NW = 32
_RPW = _B // _NW
_CHUNK = 32
_NCHUNK = _RPW // _CHUNK

_mesh = plsc.VectorSubcoreMesh(core_axis_name="c", subcore_axis_name="s")

_cp = pltpu.CompilerParams()
if "needs_layout_passes" in pltpu.CompilerParams.__dataclass_fields__:
    _cp = dataclasses.replace(_cp, needs_layout_passes=False)


@functools.partial(
    pl.kernel,
    out_type=jax.ShapeDtypeStruct((_B * _C,), jnp.float32),
    mesh=_mesh,
    scratch_types=[
        pltpu.VMEM((_CHUNK * _C,), jnp.float32),
        pltpu.VMEM((_CHUNK,), jnp.int32),
    ],
    compiler_params=_cp,
)
def _sc_onehot(lab_hbm, out_hbm, buf, lab_v):
    wid = lax.axis_index("s") * 2 + lax.axis_index("c")
    base_row = wid * _RPW

    zero16 = jnp.zeros((16,), jnp.float32)
    one16 = jnp.full((16,), 1.0, jnp.float32)
    iota16 = lax.broadcasted_iota(jnp.int32, (16,), 0)

    @pl.loop(0, _CHUNK * _C, step=16)
    def _(i):
        buf[pl.ds(i, 16)] = zero16

    @pl.loop(0, _NCHUNK)
    def _(c):
        row0 = base_row + c * _CHUNK
        pltpu.sync_copy(lab_hbm.at[pl.ds(row0, _CHUNK)], lab_v)

        @pl.loop(0, _CHUNK, step=16)
        def _(g):
            idx = (g + iota16) * _C + lab_v[pl.ds(g, 16)]
            plsc.store_scatter(buf, [idx], one16)

        pltpu.sync_copy(buf, out_hbm.at[pl.ds(row0 * _C, _CHUNK * _C)])

        @pl.loop(0, _CHUNK, step=16)
        def _(g):
            idx = (g + iota16) * _C + lab_v[pl.ds(g, 16)]
            plsc.store_scatter(buf, [idx], zero16)


def kernel(input):
    labels = input.reshape(-1)
    return _sc_onehot(labels).reshape(_B, _C)


# speedup vs baseline: 1.1199x; 1.1199x over previous
import dataclasses
import functools

import jax
import jax.numpy as jnp
from jax import lax
from jax.experimental import pallas as pl
from jax.experimental.pallas import tpu as pltpu
from jax.experimental.pallas import tpu_sc as plsc

_B = 16384
_C = 1000
_B_TC = 12288
_B_SC = _B - _B_TC

_ROWS = 4096

_NW = 32
_RPW = _B_SC // _NW
_CHUNK = 32
_NCHUNK = _RPW // _CHUNK

_mesh = plsc.VectorSubcoreMesh(core_axis_name="c", subcore_axis_name="s")

_cp = pltpu.CompilerParams()
if "needs_layout_passes" in pltpu.CompilerParams.__dataclass_fields__:
    _cp = dataclasses.replace(_cp, needs_layout_passes=False)


def _onehot_block(lab_ref, out_ref):
    labs = lab_ref[...]
    cols = jax.lax.broadcasted_iota(jnp.int32, (_ROWS, _C), 1)
    out_ref[...] = (cols == labs).astype(jnp.float32)


def _tc_onehot(labs):
    return pl.pallas_call(
        _onehot_block,
        grid=(_B_TC // _ROWS,),
        in_specs=[pl.BlockSpec((_ROWS, 1), lambda i: (i, 0))],
        out_specs=pl.BlockSpec((_ROWS, _C), lambda i: (i, 0)),
        out_shape=jax.ShapeDtypeStruct((_B_TC, _C), jnp.float32),
    )(labs)


@functools.partial(
    pl.kernel,
    out_type=jax.ShapeDtypeStruct((_B_SC * _C,), jnp.float32),
    mesh=_mesh,
    scratch_types=[
        pltpu.VMEM((_CHUNK * _C,), jnp.float32),
        pltpu.VMEM((_CHUNK,), jnp.int32),
    ],
    compiler_params=_cp,
)
def _sc_onehot(lab_hbm, out_hbm, buf, lab_v):
    wid = lax.axis_index("s") * 2 + lax.axis_index("c")
    base_row = wid * _RPW

    zero16 = jnp.zeros((16,), jnp.float32)
    one16 = jnp.full((16,), 1.0, jnp.float32)
    iota16 = lax.broadcasted_iota(jnp.int32, (16,), 0)

    @pl.loop(0, _CHUNK * _C, step=16)
    def _(i):
        buf[pl.ds(i, 16)] = zero16

    @pl.loop(0, _NCHUNK)
    def _(c):
        row0 = base_row + c * _CHUNK
        pltpu.sync_copy(lab_hbm.at[pl.ds(row0, _CHUNK)], lab_v)

        @pl.loop(0, _CHUNK, step=16)
        def _(g):
            idx = (g + iota16) * _C + lab_v[pl.ds(g, 16)]
            plsc.store_scatter(buf, [idx], one16)

        pltpu.sync_copy(buf, out_hbm.at[pl.ds(row0 * _C, _CHUNK * _C)])

        @pl.loop(0, _CHUNK, step=16)
        def _(g):
            idx = (g + iota16) * _C + lab_v[pl.ds(g, 16)]
            plsc.store_scatter(buf, [idx], zero16)


def kernel(input):
    tc_part = _tc_onehot(input[:_B_TC])
    sc_part = _sc_onehot(input[_B_TC:].reshape(-1)).reshape(_B_SC, _C)
    return jnp.concatenate([tc_part, sc_part], axis=0)


# speedup vs baseline: 1.9586x; 1.7489x over previous
import jax
import jax.numpy as jnp
from jax.experimental import pallas as pl

_LABELNUM = 1000
_ROWS = 2048


def _onehot_block(lab_ref, out_ref):
    labs = lab_ref[...]
    cols = jax.lax.broadcasted_iota(jnp.int32, (_ROWS, _LABELNUM), 1)
    out_ref[...] = (cols == labs).astype(jnp.float32)


def kernel(input):
    B, _ = input.shape
    return pl.pallas_call(
        _onehot_block,
        grid=(B // _ROWS,),
        in_specs=[pl.BlockSpec((_ROWS, 1), lambda i: (i, 0))],
        out_specs=pl.BlockSpec((_ROWS, _LABELNUM), lambda i: (i, 0)),
        out_shape=jax.ShapeDtypeStruct((B, _LABELNUM), jnp.float32),
    )(input)
